# baseline (speedup 1.0000x reference)
import functools

import numpy as np

import jax
import jax.numpy as jnp
from jax import lax
from jax.experimental import pallas as pl
from jax.experimental.pallas import tpu as pltpu
from jax.experimental.pallas import tpu_sc as plsc

B = 16384
D = 128
DP = D // 2
NC = 2
NS = 16
NW = NC * NS
ROWS_PER_W = B // NW
CHUNK = 128
NCHK = ROWS_PER_W // CHUNK
NCHK2 = 2 * NCHK

BM = 2048

_c = np.arange(64)
_idx_lo = 32 * (_c // 16) + (_c % 16)
_W1_PERM = np.concatenate(
    [_idx_lo, _idx_lo + 16, 128 + _idx_lo, 128 + _idx_lo + 16])


def _pack_chunk(src, dst):
    half = jnp.uint32(0x8000)
    himask = jnp.uint32(0xFFFF0000)

    def row(r, _):
        for g in range(4):
            lo = lax.bitcast_convert_type(src[r, pl.ds(32 * g, 16)],
                                          jnp.uint32)
            hi = lax.bitcast_convert_type(src[r, pl.ds(32 * g + 16, 16)],
                                          jnp.uint32)
            lr = lax.shift_right_logical(lo + half, jnp.uint32(16))
            hr = (hi + half) & himask
            dst[r, pl.ds(16 * g, 16)] = lax.bitcast_convert_type(
                lr | hr, jnp.int32)
        return 0

    lax.fori_loop(0, CHUNK, row, 0)


def _gather_body(user_table, item_table, uid, iid, ue_out, ie_out,
                 idx_u, idx_i, rows, packed, gsem, csem):
    wid = lax.axis_index("s") * NC + lax.axis_index("c")
    rbase = wid * ROWS_PER_W
    pltpu.sync_copy(uid.at[pl.ds(rbase, ROWS_PER_W)], idx_u)
    pltpu.sync_copy(iid.at[pl.ds(rbase, ROWS_PER_W)], idx_i)

    tabs = [user_table] * NCHK + [item_table] * NCHK
    idxs = ([idx_u.at[pl.ds(j * CHUNK, CHUNK)] for j in range(NCHK)]
            + [idx_i.at[pl.ds(j * CHUNK, CHUNK)] for j in range(NCHK)])
    outs = ([ue_out.at[pl.ds(rbase + j * CHUNK, CHUNK)] for j in range(NCHK)]
            + [ie_out.at[pl.ds(rbase + j * CHUNK, CHUNK)]
               for j in range(NCHK)])

    g = [None] * NCHK2
    c = [None] * NCHK2
    g[0] = pltpu.async_copy(tabs[0].at[idxs[0]], rows.at[0], gsem)
    g[1] = pltpu.async_copy(tabs[1].at[idxs[1]], rows.at[1], gsem)
    for k in range(NCHK2):
        g[k].wait()
        if k >= 2:
            c[k - 2].wait()
        _pack_chunk(rows.at[k % 2], packed.at[k % 2])
        c[k] = pltpu.async_copy(packed.at[k % 2], outs[k], csem)
        if k + 2 < NCHK2:
            g[k + 2] = pltpu.async_copy(
                tabs[k + 2].at[idxs[k + 2]], rows.at[k % 2], gsem)
    c[NCHK2 - 2].wait()
    c[NCHK2 - 1].wait()


@functools.cache
def _sc_gather():
    return pl.kernel(
        _gather_body,
        out_type=(
            jax.ShapeDtypeStruct((B, DP), jnp.int32),
            jax.ShapeDtypeStruct((B, DP), jnp.int32),
        ),
        mesh=plsc.VectorSubcoreMesh(core_axis_name="c", subcore_axis_name="s"),
        scratch_types=[
            pltpu.VMEM((ROWS_PER_W,), jnp.int32),
            pltpu.VMEM((ROWS_PER_W,), jnp.int32),
            pltpu.VMEM((2, CHUNK, D), jnp.float32),
            pltpu.VMEM((2, CHUNK, DP), jnp.int32),
            pltpu.SemaphoreType.DMA,
            pltpu.SemaphoreType.DMA,
        ],
    )


def _unpack(w):
    lo = lax.bitcast_convert_type(lax.shift_left(w, 16), jnp.float32)
    hi = lax.bitcast_convert_type(w & jnp.int32(-65536), jnp.float32)
    return lo, hi


def _mlp_body(ue_ref, ie_ref, w1a_ref, w1b_ref, w1c_ref, w1d_ref, b1_ref,
              w2_ref, b2_ref, w3_ref, b3_ref, out_ref):
    lo_u, hi_u = _unpack(ue_ref[...])
    lo_i, hi_i = _unpack(ie_ref[...])
    h1 = jnp.dot(lo_u, w1a_ref[...], preferred_element_type=jnp.float32)
    h1 += jnp.dot(hi_u, w1b_ref[...], preferred_element_type=jnp.float32)
    h1 += jnp.dot(lo_i, w1c_ref[...], preferred_element_type=jnp.float32)
    h1 += jnp.dot(hi_i, w1d_ref[...], preferred_element_type=jnp.float32)
    h1 = jnp.maximum(h1 + b1_ref[...].reshape(1, 128), 0.0)
    h2 = jnp.dot(h1, w2_ref[...], preferred_element_type=jnp.float32)
    h2 = jnp.maximum(h2 + b2_ref[...].reshape(1, 64), 0.0)
    r = lax.dot_general(w3_ref[...], h2, (((0,), (1,)), ((), ())),
                        preferred_element_type=jnp.float32) + b3_ref[0]
    out_ref[...] = r.reshape(1, 1, r.shape[-1])


def _mlp(ue, ie, W1p, b1, W2, b2, W3, b3):
    wspec = [pl.BlockSpec((64, 128), (lambda i, j=j: (j, 0))) for j in range(4)]
    return pl.pallas_call(
        _mlp_body,
        grid=(B // BM,),
        in_specs=[
            pl.BlockSpec((BM, DP), lambda i: (i, 0)),
            pl.BlockSpec((BM, DP), lambda i: (i, 0)),
            *wspec,
            pl.BlockSpec((128,), lambda i: (0,)),
            pl.BlockSpec((128, 64), lambda i: (0, 0)),
            pl.BlockSpec((64,), lambda i: (0,)),
            pl.BlockSpec((64, 1), lambda i: (0, 0)),
            pl.BlockSpec((1,), lambda i: (0,)),
        ],
        out_specs=pl.BlockSpec((1, 1, BM), lambda i: (i, 0, 0)),
        out_shape=jax.ShapeDtypeStruct((B // BM, 1, BM), jnp.float32),
    )(ue, ie, W1p, W1p, W1p, W1p, b1, W2, b2, W3, b3)


def kernel(user_id, item_id, user_table, item_table, W1, b1, W2, b2, W3, b3):
    ue_p, ie_p = _sc_gather()(user_table, item_table, user_id, item_id)
    W1p = jnp.take(W1, _W1_PERM, axis=0)
    out = _mlp(ue_p, ie_p, W1p, b1, W2, b2, W3, b3)
    return out.reshape(B)


# speedup vs baseline: 1.1044x; 1.1044x over previous
import functools

import jax
import jax.numpy as jnp
from jax import lax
from jax.experimental import pallas as pl
from jax.experimental.pallas import tpu as pltpu
from jax.experimental.pallas import tpu_sc as plsc

B = 16384
D = 128
NC = 2
NS = 16
NW = NC * NS
ROWS_PER_W = B // NW

BM = 4096


def _gather_body(user_table, item_table, uid, iid, ue_out, ie_out,
                 idx_u, idx_i, rows, sem):
    wid = lax.axis_index("s") * NC + lax.axis_index("c")
    rbase = wid * ROWS_PER_W
    pltpu.sync_copy(uid.at[pl.ds(rbase, ROWS_PER_W)], idx_u)
    pltpu.sync_copy(iid.at[pl.ds(rbase, ROWS_PER_W)], idx_i)
    pltpu.async_copy(user_table.at[idx_u], rows, sem).wait()
    pltpu.sync_copy(rows, ue_out.at[pl.ds(rbase, ROWS_PER_W)])
    pltpu.async_copy(item_table.at[idx_i], rows, sem).wait()
    pltpu.sync_copy(rows, ie_out.at[pl.ds(rbase, ROWS_PER_W)])


@functools.cache
def _sc_gather():
    return pl.kernel(
        _gather_body,
        out_type=(
            jax.ShapeDtypeStruct((B, D), jnp.float32),
            jax.ShapeDtypeStruct((B, D), jnp.float32),
        ),
        mesh=plsc.VectorSubcoreMesh(core_axis_name="c", subcore_axis_name="s"),
        scratch_types=[
            pltpu.VMEM((ROWS_PER_W,), jnp.int32),
            pltpu.VMEM((ROWS_PER_W,), jnp.int32),
            pltpu.VMEM((ROWS_PER_W, D), jnp.float32),
            pltpu.SemaphoreType.DMA,
        ],
    )


def _mlp_body(ue_ref, ie_ref, w1u_ref, w1i_ref, b1_ref, w2_ref, b2_ref,
              w3_ref, b3_ref, out_ref):
    h1 = jnp.dot(ue_ref[...], w1u_ref[...], preferred_element_type=jnp.float32)
    h1 += jnp.dot(ie_ref[...], w1i_ref[...], preferred_element_type=jnp.float32)
    h1 = jnp.maximum(h1 + b1_ref[...].reshape(1, 128), 0.0)
    h2 = jnp.dot(h1, w2_ref[...], preferred_element_type=jnp.float32)
    h2 = jnp.maximum(h2 + b2_ref[...].reshape(1, 64), 0.0)
    r = lax.dot_general(w3_ref[...], h2, (((0,), (1,)), ((), ())),
                        preferred_element_type=jnp.float32) + b3_ref[0]
    out_ref[...] = r.reshape(1, 1, r.shape[-1])


def _mlp(ue, ie, W1a, W1b, b1, W2, b2, W3, b3):
    return pl.pallas_call(
        _mlp_body,
        grid=(B // BM,),
        in_specs=[
            pl.BlockSpec((BM, D), lambda i: (i, 0)),
            pl.BlockSpec((BM, D), lambda i: (i, 0)),
            pl.BlockSpec((D, 128), lambda i: (0, 0)),
            pl.BlockSpec((D, 128), lambda i: (1, 0)),
            pl.BlockSpec((128,), lambda i: (0,)),
            pl.BlockSpec((128, 64), lambda i: (0, 0)),
            pl.BlockSpec((64,), lambda i: (0,)),
            pl.BlockSpec((64, 1), lambda i: (0, 0)),
            pl.BlockSpec((1,), lambda i: (0,)),
        ],
        out_specs=pl.BlockSpec((1, 1, BM), lambda i: (i, 0, 0)),
        out_shape=jax.ShapeDtypeStruct((B // BM, 1, BM), jnp.float32),
    )(ue, ie, W1a, W1b, b1, W2, b2, W3, b3)


def kernel(user_id, item_id, user_table, item_table, W1, b1, W2, b2, W3, b3):
    ue, ie = _sc_gather()(user_table, item_table, user_id, item_id)
    out = _mlp(ue, ie, W1, W1, b1, W2, b2, W3, b3)
    return out.reshape(B)
